# baseline (speedup 1.0000x reference)
import functools

import jax
import jax.numpy as jnp
from jax import lax
from jax.experimental import pallas as pl
from jax.experimental.pallas import tpu as pltpu
from jax.experimental.pallas import tpu_sc as plsc

R, C = 16384, 200
N = R * C
V = 1_000_000

NC, NS = 2, 16
NW = NC * NS
COLS = R // NW
BRW = 8
BLK = BRW * COLS
NBLK = C // BRW
STAGERS = 8
STAGE_CHUNK = V // STAGERS
STAGE_PIECE = 5000
STAGE_PIECES = STAGE_CHUNK // STAGE_PIECE
LANES = 16


def _body(x_hbm, idx_hbm, w_hbm, out_hbm,
          idx_v0, idx_v1, idx2_v0, idx2_v1, x_v0, x_v1, w_v0, w_v1,
          o_v0, o_v1, st_v0, st_v1, table,
          in_s0, in_s1, g_s0, g_s1, o_s0, o_s1,
          sti_s0, sti_s1, sto_s0, sto_s1, f_s0, f_s1):
    cid = lax.axis_index("c")
    sid = lax.axis_index("s")
    wid = sid * NC + cid
    c0 = wid * COLS

    idx_b = (idx_v0, idx_v1)
    idx2_b = (idx2_v0, idx2_v1)
    f_s = (f_s0, f_s1)
    x_b = (x_v0, x_v1)
    w_b = (w_v0, w_v1)
    o_b = (o_v0, o_v1)
    in_s = (in_s0, in_s1)
    g_s = (g_s0, g_s1)
    o_s = (o_s0, o_s1)

    def in_copies(b):
        p = b % 2
        return [
            pltpu.make_async_copy(
                x_hbm.at[pl.ds(b * BRW, BRW), pl.ds(c0, COLS)],
                x_b[p], in_s[p]),
            pltpu.make_async_copy(
                idx_hbm.at[pl.ds(b * BRW, BRW), pl.ds(c0, COLS)],
                idx2_b[p], in_s[p]),
        ]

    def flatten(b):
        p = b % 2

        def step(i, c):
            r = i // (COLS // LANES)
            sl = pl.ds((i % (COLS // LANES)) * LANES, LANES)
            idx_b[p][pl.ds(i * LANES, LANES)] = idx2_b[p][r, sl]
            return c

        lax.fori_loop(0, BLK // LANES, step, 0, unroll=8)

    def gather_copies(b):
        p = b % 2
        return [pltpu.make_async_copy(table.at[idx_b[p]], w_b[p], g_s[p])]

    def out_copies(b):
        p = b % 2
        return [pltpu.make_async_copy(
            o_b[p], out_hbm.at[pl.ds(b * BRW, BRW), pl.ds(c0, COLS)],
            o_s[p])]

    for c in in_copies(0):
        c.start()
    for c in in_copies(1):
        c.start()

    st_b = (st_v0, st_v1)
    sti_s = (sti_s0, sti_s1)
    sto_s = (sto_s0, sto_s1)

    def stage_in(k):
        off = sid * STAGE_CHUNK + k * STAGE_PIECE
        p = k % 2
        return pltpu.make_async_copy(
            w_hbm.at[pl.ds(off, STAGE_PIECE)], st_b[p], sti_s[p])

    def stage_out(k):
        off = sid * STAGE_CHUNK + k * STAGE_PIECE
        p = k % 2
        return pltpu.make_async_copy(
            st_b[p], table.at[pl.ds(off, STAGE_PIECE)], sto_s[p])

    @pl.when(sid < STAGERS)
    def _():
        stage_in(0).start()
        for k in range(STAGE_PIECES):
            if k >= 1:
                stage_out(k - 1).wait()
            if k + 1 < STAGE_PIECES:
                stage_in(k + 1).start()
            stage_in(k).wait()
            stage_out(k).start()
        stage_out(STAGE_PIECES - 1).wait()

    plsc.subcore_barrier()

    for c in in_copies(0):
        c.wait()
    flatten(0)
    for c in gather_copies(0):
        c.start()

    def mul(b):
        p = b % 2

        def step(i, c):
            r = i // (COLS // LANES)
            sl = pl.ds((i % (COLS // LANES)) * LANES, LANES)
            o_b[p][r, sl] = x_b[p][r, sl] * w_b[p][pl.ds(i * LANES, LANES)]
            return c

        lax.fori_loop(0, BLK // LANES, step, 0, unroll=8)

    for b in range(NBLK):
        if b + 1 < NBLK:
            for c in in_copies(b + 1):
                c.wait()
            flatten(b + 1)
            for c in gather_copies(b + 1):
                c.start()
        for c in gather_copies(b):
            c.wait()
        if b >= 2:
            for c in out_copies(b - 2):
                c.wait()
        mul(b)
        for c in out_copies(b):
            c.start()
        if b + 2 < NBLK:
            for c in in_copies(b + 2):
                c.start()

    for c in out_copies(NBLK - 2):
        c.wait()
    for c in out_copies(NBLK - 1):
        c.wait()


@jax.jit
def kernel(x, index, weight):
    mesh = plsc.VectorSubcoreMesh(core_axis_name="c", subcore_axis_name="s")
    run = functools.partial(
        pl.kernel,
        mesh=mesh,
        out_type=jax.ShapeDtypeStruct((C, R), jnp.float32),
        scratch_types=[
            pltpu.VMEM((BLK,), jnp.int32),
            pltpu.VMEM((BLK,), jnp.int32),
            pltpu.VMEM((BRW, COLS), jnp.int32),
            pltpu.VMEM((BRW, COLS), jnp.int32),
            pltpu.VMEM((BRW, COLS), jnp.float32),
            pltpu.VMEM((BRW, COLS), jnp.float32),
            pltpu.VMEM((BLK,), jnp.float32),
            pltpu.VMEM((BLK,), jnp.float32),
            pltpu.VMEM((BRW, COLS), jnp.float32),
            pltpu.VMEM((BRW, COLS), jnp.float32),
            pltpu.VMEM((STAGE_PIECE,), jnp.float32),
            pltpu.VMEM((STAGE_PIECE,), jnp.float32),
            pltpu.VMEM_SHARED((V,), jnp.float32),
            pltpu.SemaphoreType.DMA,
            pltpu.SemaphoreType.DMA,
            pltpu.SemaphoreType.DMA,
            pltpu.SemaphoreType.DMA,
            pltpu.SemaphoreType.DMA,
            pltpu.SemaphoreType.DMA,
            pltpu.SemaphoreType.DMA,
            pltpu.SemaphoreType.DMA,
            pltpu.SemaphoreType.DMA,
            pltpu.SemaphoreType.DMA,
            pltpu.SemaphoreType.DMA,
            pltpu.SemaphoreType.DMA,
        ],
        compiler_params=pltpu.CompilerParams(use_tc_tiling_on_sc=True),
    )(_body)
    out_t = run(x.T, index.T.astype(jnp.int32), weight)
    return out_t.T


# speedup vs baseline: 1.1990x; 1.1990x over previous
import functools

import jax
import jax.numpy as jnp
from jax import lax
from jax.experimental import pallas as pl
from jax.experimental.pallas import tpu as pltpu
from jax.experimental.pallas import tpu_sc as plsc

R, C = 16384, 200
N = R * C
V = 1_000_000

NC, NS = 2, 16
NW = NC * NS
COLS = R // NW
BRW = 8
BLK = BRW * COLS
NBLK = C // BRW
STAGERS = 8
STAGE_CHUNK = V // STAGERS
STAGE_PIECE = 5000
STAGE_PIECES = STAGE_CHUNK // STAGE_PIECE
LANES = 16


def _body(x_hbm, idx_hbm, w_hbm, out_hbm,
          idx_v0, idx_v1, x_v0, x_v1, w_v0, w_v1, o_v0, o_v1,
          st_v0, st_v1, table,
          in_s0, in_s1, g_s0, g_s1, o_s0, o_s1,
          sti_s0, sti_s1, sto_s0, sto_s1):
    cid = lax.axis_index("c")
    sid = lax.axis_index("s")
    wid = sid * NC + cid
    c0 = wid * COLS

    idx_b = (idx_v0, idx_v1)
    x_b = (x_v0, x_v1)
    w_b = (w_v0, w_v1)
    o_b = (o_v0, o_v1)
    in_s = (in_s0, in_s1)
    g_s = (g_s0, g_s1)
    o_s = (o_s0, o_s1)

    def in_copies(b):
        p = b % 2
        cs = [pltpu.make_async_copy(
            x_hbm.at[pl.ds(b * BRW, BRW), pl.ds(c0, COLS)],
            x_b[p], in_s[p])]
        for r in range(BRW):
            row = b * BRW + r
            cs.append(pltpu.make_async_copy(
                idx_hbm.at[row, pl.ds(c0, COLS)],
                idx_b[p].at[pl.ds(r * COLS, COLS)], in_s[p]))
        return cs

    def gather_copies(b):
        p = b % 2
        return [pltpu.make_async_copy(table.at[idx_b[p]], w_b[p], g_s[p])]

    def out_copies(b):
        p = b % 2
        return [pltpu.make_async_copy(
            o_b[p], out_hbm.at[pl.ds(b * BRW, BRW), pl.ds(c0, COLS)],
            o_s[p])]

    for c in in_copies(0):
        c.start()
    for c in in_copies(1):
        c.start()

    st_b = (st_v0, st_v1)
    sti_s = (sti_s0, sti_s1)
    sto_s = (sto_s0, sto_s1)

    def stage_in(k):
        off = sid * STAGE_CHUNK + k * STAGE_PIECE
        p = k % 2
        return pltpu.make_async_copy(
            w_hbm.at[pl.ds(off, STAGE_PIECE)], st_b[p], sti_s[p])

    def stage_out(k):
        off = sid * STAGE_CHUNK + k * STAGE_PIECE
        p = k % 2
        return pltpu.make_async_copy(
            st_b[p], table.at[pl.ds(off, STAGE_PIECE)], sto_s[p])

    @pl.when(sid < STAGERS)
    def _():
        stage_in(0).start()
        for k in range(STAGE_PIECES):
            if k >= 1:
                stage_out(k - 1).wait()
            if k + 1 < STAGE_PIECES:
                stage_in(k + 1).start()
            stage_in(k).wait()
            stage_out(k).start()
        stage_out(STAGE_PIECES - 1).wait()

    plsc.subcore_barrier()

    for c in in_copies(0):
        c.wait()
    for c in gather_copies(0):
        c.start()

    def mul(b):
        p = b % 2

        def step(i, c):
            r = i // (COLS // LANES)
            sl = pl.ds((i % (COLS // LANES)) * LANES, LANES)
            o_b[p][r, sl] = x_b[p][r, sl] * w_b[p][pl.ds(i * LANES, LANES)]
            return c

        lax.fori_loop(0, BLK // LANES, step, 0, unroll=8)

    for b in range(NBLK):
        if b + 1 < NBLK:
            for c in in_copies(b + 1):
                c.wait()
            for c in gather_copies(b + 1):
                c.start()
        for c in gather_copies(b):
            c.wait()
        if b >= 2:
            for c in out_copies(b - 2):
                c.wait()
        mul(b)
        for c in out_copies(b):
            c.start()
        if b + 2 < NBLK:
            for c in in_copies(b + 2):
                c.start()

    for c in out_copies(NBLK - 2):
        c.wait()
    for c in out_copies(NBLK - 1):
        c.wait()


@jax.jit
def kernel(x, index, weight):
    mesh = plsc.VectorSubcoreMesh(core_axis_name="c", subcore_axis_name="s")
    run = functools.partial(
        pl.kernel,
        mesh=mesh,
        out_type=jax.ShapeDtypeStruct((C, R), jnp.float32),
        scratch_types=[
            pltpu.VMEM((BLK,), jnp.int32),
            pltpu.VMEM((BLK,), jnp.int32),
            pltpu.VMEM((BRW, COLS), jnp.float32),
            pltpu.VMEM((BRW, COLS), jnp.float32),
            pltpu.VMEM((BLK,), jnp.float32),
            pltpu.VMEM((BLK,), jnp.float32),
            pltpu.VMEM((BRW, COLS), jnp.float32),
            pltpu.VMEM((BRW, COLS), jnp.float32),
            pltpu.VMEM((STAGE_PIECE,), jnp.float32),
            pltpu.VMEM((STAGE_PIECE,), jnp.float32),
            pltpu.VMEM_SHARED((V,), jnp.float32),
            pltpu.SemaphoreType.DMA,
            pltpu.SemaphoreType.DMA,
            pltpu.SemaphoreType.DMA,
            pltpu.SemaphoreType.DMA,
            pltpu.SemaphoreType.DMA,
            pltpu.SemaphoreType.DMA,
            pltpu.SemaphoreType.DMA,
            pltpu.SemaphoreType.DMA,
            pltpu.SemaphoreType.DMA,
            pltpu.SemaphoreType.DMA,
        ],
        compiler_params=pltpu.CompilerParams(use_tc_tiling_on_sc=True),
    )(_body)
    out_t = run(x.T, index.T.astype(jnp.int32), weight)
    return out_t.T


# speedup vs baseline: 1.2808x; 1.0682x over previous
import functools

import jax
import jax.numpy as jnp
from jax import lax
from jax.experimental import pallas as pl
from jax.experimental.pallas import tpu as pltpu
from jax.experimental.pallas import tpu_sc as plsc

R, C = 16384, 200
N = R * C
V = 1_000_000

NC, NS = 2, 16
NW = NC * NS
COLS = R // NW
BRW = 8
BLK = BRW * COLS
NBLK = C // BRW
STAGE_PIECE = 5000
STAGE_PIECES_TOT = V // STAGE_PIECE
STAGE_FULL = STAGE_PIECES_TOT // NS
STAGE_TAIL = STAGE_PIECES_TOT - STAGE_FULL * NS
LANES = 16


def _body(x_hbm, idx_hbm, w_hbm, out_hbm,
          idx_v0, idx_v1, x_v0, x_v1, w_v0, w_v1, o_v0, o_v1,
          st_v0, st_v1, table,
          in_s0, in_s1, g_s0, g_s1, o_s0, o_s1,
          sti_s0, sti_s1, sto_s0, sto_s1):
    cid = lax.axis_index("c")
    sid = lax.axis_index("s")
    wid = sid * NC + cid
    c0 = wid * COLS

    idx_b = (idx_v0, idx_v1)
    x_b = (x_v0, x_v1)
    w_b = (w_v0, w_v1)
    o_b = (o_v0, o_v1)
    in_s = (in_s0, in_s1)
    g_s = (g_s0, g_s1)
    o_s = (o_s0, o_s1)

    def in_copies(b):
        p = b % 2
        cs = [pltpu.make_async_copy(
            x_hbm.at[pl.ds(b * BRW, BRW), pl.ds(c0, COLS)],
            x_b[p], in_s[p])]
        for r in range(BRW):
            row = b * BRW + r
            cs.append(pltpu.make_async_copy(
                idx_hbm.at[row, pl.ds(c0, COLS)],
                idx_b[p].at[pl.ds(r * COLS, COLS)], in_s[p]))
        return cs

    def gather_copies(b):
        p = b % 2
        return [pltpu.make_async_copy(table.at[idx_b[p]], w_b[p], g_s[p])]

    def out_copies(b):
        p = b % 2
        return [pltpu.make_async_copy(
            o_b[p], out_hbm.at[pl.ds(b * BRW, BRW), pl.ds(c0, COLS)],
            o_s[p])]

    for c in in_copies(0):
        c.start()
    for c in in_copies(1):
        c.start()

    st_b = (st_v0, st_v1)
    sti_s = (sti_s0, sti_s1)
    sto_s = (sto_s0, sto_s1)

    def stage_in(k):
        off = (k * NS + sid) * STAGE_PIECE
        p = k % 2
        return pltpu.make_async_copy(
            w_hbm.at[pl.ds(off, STAGE_PIECE)], st_b[p], sti_s[p])

    def stage_out(k):
        off = (k * NS + sid) * STAGE_PIECE
        p = k % 2
        return pltpu.make_async_copy(
            st_b[p], table.at[pl.ds(off, STAGE_PIECE)], sto_s[p])

    stage_in(0).start()
    for k in range(STAGE_FULL):
        if k >= 1:
            stage_out(k - 1).wait()
        if k + 1 < STAGE_FULL:
            stage_in(k + 1).start()
        elif STAGE_TAIL:
            @pl.when(sid < STAGE_TAIL)
            def _():
                stage_in(STAGE_FULL).start()
        stage_in(k).wait()
        stage_out(k).start()
    stage_out(STAGE_FULL - 1).wait()
    if STAGE_TAIL:
        @pl.when(sid < STAGE_TAIL)
        def _():
            stage_in(STAGE_FULL).wait()
            stage_out(STAGE_FULL).start()
            stage_out(STAGE_FULL).wait()

    plsc.subcore_barrier()

    for c in in_copies(0):
        c.wait()
    for c in gather_copies(0):
        c.start()

    def mul(b):
        p = b % 2

        def step(i, c):
            r = i // (COLS // LANES)
            sl = pl.ds((i % (COLS // LANES)) * LANES, LANES)
            o_b[p][r, sl] = x_b[p][r, sl] * w_b[p][pl.ds(i * LANES, LANES)]
            return c

        lax.fori_loop(0, BLK // LANES, step, 0, unroll=8)

    for b in range(NBLK):
        if b + 1 < NBLK:
            for c in in_copies(b + 1):
                c.wait()
            for c in gather_copies(b + 1):
                c.start()
        for c in gather_copies(b):
            c.wait()
        if b >= 2:
            for c in out_copies(b - 2):
                c.wait()
        mul(b)
        for c in out_copies(b):
            c.start()
        if b + 2 < NBLK:
            for c in in_copies(b + 2):
                c.start()

    for c in out_copies(NBLK - 2):
        c.wait()
    for c in out_copies(NBLK - 1):
        c.wait()


@jax.jit
def kernel(x, index, weight):
    mesh = plsc.VectorSubcoreMesh(core_axis_name="c", subcore_axis_name="s")
    run = functools.partial(
        pl.kernel,
        mesh=mesh,
        out_type=jax.ShapeDtypeStruct((C, R), jnp.float32),
        scratch_types=[
            pltpu.VMEM((BLK,), jnp.int32),
            pltpu.VMEM((BLK,), jnp.int32),
            pltpu.VMEM((BRW, COLS), jnp.float32),
            pltpu.VMEM((BRW, COLS), jnp.float32),
            pltpu.VMEM((BLK,), jnp.float32),
            pltpu.VMEM((BLK,), jnp.float32),
            pltpu.VMEM((BRW, COLS), jnp.float32),
            pltpu.VMEM((BRW, COLS), jnp.float32),
            pltpu.VMEM((STAGE_PIECE,), jnp.float32),
            pltpu.VMEM((STAGE_PIECE,), jnp.float32),
            pltpu.VMEM_SHARED((V,), jnp.float32),
            pltpu.SemaphoreType.DMA,
            pltpu.SemaphoreType.DMA,
            pltpu.SemaphoreType.DMA,
            pltpu.SemaphoreType.DMA,
            pltpu.SemaphoreType.DMA,
            pltpu.SemaphoreType.DMA,
            pltpu.SemaphoreType.DMA,
            pltpu.SemaphoreType.DMA,
            pltpu.SemaphoreType.DMA,
            pltpu.SemaphoreType.DMA,
        ],
        compiler_params=pltpu.CompilerParams(use_tc_tiling_on_sc=True),
    )(_body)
    out_t = run(x.T, index.T.astype(jnp.int32), weight)
    return out_t.T


# speedup vs baseline: 1.2996x; 1.0147x over previous
import functools

import jax
import jax.numpy as jnp
from jax import lax
from jax.experimental import pallas as pl
from jax.experimental.pallas import tpu as pltpu
from jax.experimental.pallas import tpu_sc as plsc

R, C = 16384, 200
N = R * C
V = 1_000_000

NC, NS = 2, 16
NW = NC * NS
COLS = R // NW
BRW = 8
BLK = BRW * COLS
NBLK = C // BRW
STAGE_PIECE = 8000
STAGE_PIECES_TOT = V // STAGE_PIECE
STAGE_FULL = STAGE_PIECES_TOT // NS
STAGE_TAIL = STAGE_PIECES_TOT - STAGE_FULL * NS
LANES = 16


def _body(x_hbm, idx_hbm, w_hbm, out_hbm,
          idx_v0, idx_v1, x_v0, x_v1, w_v0, w_v1, o_v0, o_v1,
          st_v0, st_v1, table,
          in_s0, in_s1, g_s0, g_s1, o_s0, o_s1,
          sti_s0, sti_s1, sto_s0, sto_s1):
    cid = lax.axis_index("c")
    sid = lax.axis_index("s")
    wid = sid * NC + cid
    c0 = wid * COLS

    idx_b = (idx_v0, idx_v1)
    x_b = (x_v0, x_v1)
    w_b = (w_v0, w_v1)
    o_b = (o_v0, o_v1)
    in_s = (in_s0, in_s1)
    g_s = (g_s0, g_s1)
    o_s = (o_s0, o_s1)

    def in_copies(b):
        p = b % 2
        cs = [pltpu.make_async_copy(
            x_hbm.at[pl.ds(b * BRW, BRW), pl.ds(c0, COLS)],
            x_b[p], in_s[p])]
        for r in range(BRW):
            row = b * BRW + r
            cs.append(pltpu.make_async_copy(
                idx_hbm.at[row, pl.ds(c0, COLS)],
                idx_b[p].at[pl.ds(r * COLS, COLS)], in_s[p]))
        return cs

    def gather_copies(b):
        p = b % 2
        return [pltpu.make_async_copy(table.at[idx_b[p]], w_b[p], g_s[p])]

    def out_copies(b):
        p = b % 2
        return [pltpu.make_async_copy(
            o_b[p], out_hbm.at[pl.ds(b * BRW, BRW), pl.ds(c0, COLS)],
            o_s[p])]

    for c in in_copies(0):
        c.start()
    for c in in_copies(1):
        c.start()

    st_b = (st_v0, st_v1)
    sti_s = (sti_s0, sti_s1)
    sto_s = (sto_s0, sto_s1)

    def stage_in(k):
        off = (k * NS + sid) * STAGE_PIECE
        p = k % 2
        return pltpu.make_async_copy(
            w_hbm.at[pl.ds(off, STAGE_PIECE)], st_b[p], sti_s[p])

    def stage_out(k):
        off = (k * NS + sid) * STAGE_PIECE
        p = k % 2
        return pltpu.make_async_copy(
            st_b[p], table.at[pl.ds(off, STAGE_PIECE)], sto_s[p])

    stage_in(0).start()
    for k in range(STAGE_FULL):
        if k >= 1:
            stage_out(k - 1).wait()
        if k + 1 < STAGE_FULL:
            stage_in(k + 1).start()
        elif STAGE_TAIL:
            @pl.when(sid < STAGE_TAIL)
            def _():
                stage_in(STAGE_FULL).start()
        stage_in(k).wait()
        stage_out(k).start()
    stage_out(STAGE_FULL - 1).wait()
    if STAGE_TAIL:
        @pl.when(sid < STAGE_TAIL)
        def _():
            stage_in(STAGE_FULL).wait()
            stage_out(STAGE_FULL).start()
            stage_out(STAGE_FULL).wait()

    plsc.subcore_barrier()

    for c in in_copies(0):
        c.wait()
    for c in gather_copies(0):
        c.start()

    def mul(b):
        p = b % 2

        def step(i, c):
            r = i // (COLS // LANES)
            sl = pl.ds((i % (COLS // LANES)) * LANES, LANES)
            o_b[p][r, sl] = x_b[p][r, sl] * w_b[p][pl.ds(i * LANES, LANES)]
            return c

        lax.fori_loop(0, BLK // LANES, step, 0, unroll=8)

    for b in range(NBLK):
        if b + 1 < NBLK:
            for c in in_copies(b + 1):
                c.wait()
            for c in gather_copies(b + 1):
                c.start()
        for c in gather_copies(b):
            c.wait()
        if b >= 2:
            for c in out_copies(b - 2):
                c.wait()
        mul(b)
        for c in out_copies(b):
            c.start()
        if b + 2 < NBLK:
            for c in in_copies(b + 2):
                c.start()

    for c in out_copies(NBLK - 2):
        c.wait()
    for c in out_copies(NBLK - 1):
        c.wait()


@jax.jit
def kernel(x, index, weight):
    mesh = plsc.VectorSubcoreMesh(core_axis_name="c", subcore_axis_name="s")
    run = functools.partial(
        pl.kernel,
        mesh=mesh,
        out_type=jax.ShapeDtypeStruct((C, R), jnp.float32),
        scratch_types=[
            pltpu.VMEM((BLK,), jnp.int32),
            pltpu.VMEM((BLK,), jnp.int32),
            pltpu.VMEM((BRW, COLS), jnp.float32),
            pltpu.VMEM((BRW, COLS), jnp.float32),
            pltpu.VMEM((BLK,), jnp.float32),
            pltpu.VMEM((BLK,), jnp.float32),
            pltpu.VMEM((BRW, COLS), jnp.float32),
            pltpu.VMEM((BRW, COLS), jnp.float32),
            pltpu.VMEM((STAGE_PIECE,), jnp.float32),
            pltpu.VMEM((STAGE_PIECE,), jnp.float32),
            pltpu.VMEM_SHARED((V,), jnp.float32),
            pltpu.SemaphoreType.DMA,
            pltpu.SemaphoreType.DMA,
            pltpu.SemaphoreType.DMA,
            pltpu.SemaphoreType.DMA,
            pltpu.SemaphoreType.DMA,
            pltpu.SemaphoreType.DMA,
            pltpu.SemaphoreType.DMA,
            pltpu.SemaphoreType.DMA,
            pltpu.SemaphoreType.DMA,
            pltpu.SemaphoreType.DMA,
        ],
        compiler_params=pltpu.CompilerParams(use_tc_tiling_on_sc=True),
    )(_body)
    out_t = run(x.T, index.T.astype(jnp.int32), weight)
    return out_t.T
